# baseline (speedup 1.0000x reference)
import jax
import jax.numpy as jnp
from jax.experimental import pallas as pl

B = 128
H = 2048
I = 1024
K = 8
LOCAL = 8
NSPLIT = 2
IS = I // NSPLIT


def _moe_body(x_ref, w1g_ref, w1u_ref, w2_ref, eid_ref, sc_ref, out_ref):
    e = pl.program_id(0)
    j = pl.program_id(1)
    f32 = jnp.float32
    bf16 = jnp.bfloat16
    xb = x_ref[...].astype(bf16)
    gate = jnp.dot(xb, w1g_ref[0].astype(bf16), preferred_element_type=f32)
    up = jnp.dot(xb, w1u_ref[0].astype(bf16), preferred_element_type=f32)
    w = jnp.sum(jnp.where(eid_ref[...] == e, sc_ref[...], 0.0), axis=1)
    a = gate * jax.nn.sigmoid(gate) * up * w[:, None]

    @pl.when(jnp.logical_and(e == 0, j == 0))
    def _():
        out_ref[...] = jnp.zeros_like(out_ref)

    out_ref[...] += jnp.dot(a.astype(bf16), w2_ref[0].astype(bf16),
                            preferred_element_type=f32)


def kernel(x, expert_ids, smooth_scales, expert_scales, x_active_mask,
           gmm1_weight, gmm2_weight):
    del smooth_scales
    eids = expert_ids.astype(jnp.int32)
    sc = expert_scales * x_active_mask[:, None].astype(jnp.float32)

    out = pl.pallas_call(
        _moe_body,
        grid=(LOCAL, NSPLIT),
        in_specs=[
            pl.BlockSpec((B, H), lambda e, j: (0, 0)),
            pl.BlockSpec((1, H, IS), lambda e, j: (e, 0, j)),
            pl.BlockSpec((1, H, IS), lambda e, j: (e, 0, NSPLIT + j)),
            pl.BlockSpec((1, IS, H), lambda e, j: (e, j, 0)),
            pl.BlockSpec((B, K), lambda e, j: (0, 0)),
            pl.BlockSpec((B, K), lambda e, j: (0, 0)),
        ],
        out_specs=pl.BlockSpec((B, H), lambda e, j: (0, 0)),
        out_shape=jax.ShapeDtypeStruct((B, H), jnp.float32),
    )(x, gmm1_weight, gmm1_weight, gmm2_weight, eids, sc)
    return out
